# baseline (speedup 1.0000x reference)
import functools

import jax
import jax.numpy as jnp
from jax import lax
from jax.experimental import pallas as pl
from jax.experimental.pallas import tpu as pltpu
from jax.experimental.pallas import tpu_sc as plsc

B = 16384
D = 32
H = 256
NT = 6
NBIG = 2
NSML = 4
NC, NS = 2, 16
NW = NC * NS
BPW = B // NW
CH = 128
NCH = BPW // CH

VBIG = 1000000
TILES = (VBIG + 127) // 128
TPW = (TILES + NW - 1) // NW
CHT = 6
CLANES = CHT * 128
CAP = 768
LCAP = 800
CAPP = 832
HDR = CAPP - 1
CAPF = CAPP * D // 128
SENT = B
BD = B + 2048


def _sc_scan_big(cat0, cat1, tt0, tt1):
  mesh = plsc.VectorSubcoreMesh(core_axis_name="c", subcore_axis_name="s")

  @functools.partial(
      pl.kernel,
      out_type=(jax.ShapeDtypeStruct((NBIG, NW, CAPF, 128), jnp.float32),
                jax.ShapeDtypeStruct((NBIG, NW, CAPP), jnp.int32)),
      mesh=mesh,
      compiler_params=pltpu.CompilerParams(
          use_tc_tiling_on_sc=True, needs_layout_passes=False),
      scratch_types=[
          pltpu.VMEM((B // 4,), jnp.int32),
          pltpu.VMEM((LCAP + 16,), jnp.int32),
          pltpu.VMEM((LCAP + 16,), jnp.int32),
          pltpu.VMEM((LCAP + 16,), jnp.int32),
          pltpu.VMEM((LCAP + 16,), jnp.int32),
          pltpu.VMEM((2, 4, 8, CLANES), jnp.float32),
          pltpu.VMEM((CAPF, 128), jnp.float32),
          pltpu.VMEM((CAPP,), jnp.int32),
          pltpu.SemaphoreType.DMA,
      ],
  )
  def k(c0, c1, t0, t1, out, opos, idx_v, vlist, plist, clist, cplist,
        dbuf, packed_v, pos_out, sem):
    wid = lax.axis_index("s") * NC + lax.axis_index("c")
    d16 = lax.iota(jnp.int32, 16)
    ga, ra = d16 // 8, d16 % 8
    gb, rb = ga + 2, ra
    lo = wid * TPW
    lo = jnp.minimum(lo, TILES - CHT)
    hi = jnp.minimum(wid * TPW + TPW, TILES)
    nch = (hi - lo + CHT - 1) // CHT
    lov = jnp.full((16,), lo * 128, jnp.int32)
    hiv = jnp.full((16,), wid * TPW * 128 + TPW * 128, jnp.int32)

    for t, (cat, tab) in enumerate(((c0, t0), (c1, t1))):
      for m in range(0, LCAP + 16, 16):
        vlist[pl.ds(m, 16)] = jnp.full((16,), -1, jnp.int32)

      cnt = 0
      for q in range(4):
        pltpu.sync_copy(cat.at[pl.ds(q * (B // 4), B // 4)], idx_v)

        def scan(i, cnt, q=q):
          v16 = idx_v[pl.ds(i * 16, 16)]
          mask = (v16 >= lov) & (v16 < hiv)
          c = jnp.minimum(cnt, CAP)
          plsc.store_compressed(vlist.at[pl.ds(c, 16)], v16, mask=mask)
          plsc.store_compressed(plist.at[pl.ds(c, 16)],
                                d16 + i * 16 + q * (B // 4), mask=mask)
          return cnt + plsc.all_reduce_population_count(mask)[0]

        cnt = lax.fori_loop(0, B // 64, scan, cnt)
      cnt = jnp.minimum(cnt, CAP)
      nvec = (cnt + 15) // 16

      def fire(j, tab=tab):
        start = jnp.minimum(lo + j * CHT, hi - CHT)
        pltpu.async_copy(
            tab.at[:, :, pl.ds(start * 128, CLANES)],
            dbuf.at[lax.rem(j, 2)], sem)

      fire(0)

      def chunk(j, mtot, tab=tab):
        slot = lax.rem(j, 2)
        pltpu.make_async_copy(
            tab.at[:, :, pl.ds(0, CLANES)], dbuf.at[slot], sem).wait()

        @pl.when(j + 1 < nch)
        def _():
          fire(j + 1)

        start = jnp.minimum(lo + j * CHT, hi - CHT)
        b0 = jnp.full((16,), start * 128, jnp.int32)
        b1 = b0 + CLANES

        def rescan(m, mcnt):
          v16 = vlist[pl.ds(m * 16, 16)]
          mask = (v16 >= b0) & (v16 < b1)
          plsc.store_compressed(clist.at[pl.ds(mcnt, 16)], v16, mask=mask)
          plsc.store_compressed(cplist.at[pl.ds(mcnt, 16)],
                                plist[pl.ds(m * 16, 16)], mask=mask)
          return mcnt + plsc.all_reduce_population_count(mask)[0]

        mcnt = lax.fori_loop(0, nvec, rescan, 0)

        def extract(n, _):
          v = clist[pl.ds(n, 16)][0]
          p = cplist[pl.ds(n, 16)][0]
          lane = jnp.full((16,), v - start * 128, jnp.int32)
          rs = jnp.minimum(mtot + n, CAPP - 2)
          fa = rs * D + d16
          fb = fa + 16
          va = plsc.load_gather(dbuf.at[slot], [ga, ra, lane])
          vb = plsc.load_gather(dbuf.at[slot], [gb, rb, lane])
          plsc.store_scatter(packed_v, [fa >> 7, fa & 127], va)
          plsc.store_scatter(packed_v, [fb >> 7, fb & 127], vb)
          plsc.store_scatter(pos_out, [jnp.full((16,), rs, jnp.int32)],
                             jnp.full((16,), p, jnp.int32), mask=d16 < 1)
          return 0

        lax.fori_loop(0, mcnt, extract, 0)
        return mtot + mcnt

      mtot = lax.fori_loop(0, nch, chunk, 0)
      plsc.store_scatter(pos_out, [jnp.full((16,), HDR, jnp.int32)],
                         jnp.full((16,), jnp.minimum(mtot, HDR), jnp.int32),
                         mask=d16 < 1)
      pltpu.sync_copy(packed_v, out.at[t, wid])
      pltpu.sync_copy(pos_out, opos.at[t, wid])

  return k(cat0, cat1, tt0, tt1)


def _sc_small_and_scatter(cats2d, tables, pk32, pos):
  mesh = plsc.VectorSubcoreMesh(core_axis_name="c", subcore_axis_name="s")
  nseg = (CAPP + 127) // 128

  @functools.partial(
      pl.kernel,
      out_type=(jax.ShapeDtypeStruct((NSML, B, D), jnp.float32),
                jax.ShapeDtypeStruct((NBIG, BD, D), jnp.float32)),
      mesh=mesh,
      compiler_params=pltpu.CompilerParams(
          use_tc_tiling_on_sc=False, needs_layout_passes=False),
      scratch_types=[
          pltpu.VMEM((NSML * NCH, CH), jnp.int32),
          pltpu.VMEM((NSML, BPW, D), jnp.float32),
          pltpu.VMEM((nseg * 128, D), jnp.float32),
          pltpu.VMEM((nseg * 128 + 16,), jnp.int32),
          pltpu.VMEM((nseg, 128), jnp.int32),
          pltpu.SemaphoreType.DMA,
      ],
  )
  def k(c0, c1, c2, c3, t0, t1, t2, t3, pk, po, out, out2,
        idx_v, rows_v, pk_v, pos_v, plist2, sem):
    wid = lax.axis_index("s") * NC + lax.axis_index("c")
    base = wid * BPW
    d16 = lax.iota(jnp.int32, 16)
    cats = [c0, c1, c2, c3]
    tabs = [t0, t1, t2, t3]
    copies = []
    for i in range(NSML):
      pltpu.sync_copy(cats[i].at[pl.ds(wid * NCH, NCH)],
                      idx_v.at[pl.ds(i * NCH, NCH)])
      for j in range(NCH):
        copies.append(pltpu.async_copy(
            tabs[i].at[idx_v.at[i * NCH + j]],
            rows_v.at[i, pl.ds(j * CH, CH)], sem))
    for i in range(NSML):
      for j in range(NCH):
        copies[i * NCH + j].wait()
      pltpu.sync_copy(rows_v.at[i], out.at[i, pl.ds(base, BPW)])

    for t in range(NBIG):
      pltpu.sync_copy(pk.at[t, wid], pk_v.at[pl.ds(0, CAPP)])
      pltpu.sync_copy(po.at[t, wid], pos_v.at[pl.ds(0, CAPP)])
      cnt = jnp.full((16,), pos_v[pl.ds(HDR - 7, 16)][7], jnp.int32)
      for m in range(nseg * 8):
        r16 = d16 + m * 16
        pos16 = pos_v[pl.ds(m * 16, 16)]
        valid = (r16 < cnt) & (r16 < HDR)
        psel = jnp.where(valid, pos16, SENT)
        psel = jnp.where((psel >= 0) & (psel < B), psel, SENT)
        plsc.store_scatter(plist2, [r16 >> 7, r16 & 127], psel)
      for s in range(nseg):
        copies2 = pltpu.async_copy(
            pk_v.at[pl.ds(s * 128, 128)],
            out2.at[t].at[plist2.at[s]], sem)
        copies2.wait()

  return k(*cats2d, *tables, pk32, pos)


def _mlp_body(xb_ref, xs_ref, n_ref, w1b_ref, w1s_ref, wa_ref, w2_ref,
              b2_ref, o_ref):
  h = jnp.dot(n_ref[...], wa_ref[...], preferred_element_type=jnp.float32)
  for t in range(NBIG):
    h = h + jnp.dot(xb_ref[t], w1b_ref[t],
                    preferred_element_type=jnp.float32)
  for t in range(NSML):
    h = h + jnp.dot(xs_ref[t], w1s_ref[t],
                    preferred_element_type=jnp.float32)
  h = jnp.maximum(h, 0.0)
  o_ref[...] = (
      jnp.dot(h, w2_ref[...], preferred_element_type=jnp.float32)
      + b2_ref[...]
  )


def _tc_mlp(xb, xs, num_aug, w1b, w1s, w1_aug, w2, b2_2d):
  blk = 2048
  nb = B // blk
  return pl.pallas_call(
      _mlp_body,
      grid=(nb,),
      in_specs=[
          pl.BlockSpec((NBIG, blk, D), lambda i: (0, i, 0)),
          pl.BlockSpec((NSML, blk, D), lambda i: (0, i, 0)),
          pl.BlockSpec((blk, 8), lambda i: (i, 0)),
          pl.BlockSpec((NBIG, D, H), lambda i: (0, 0, 0)),
          pl.BlockSpec((NSML, D, H), lambda i: (0, 0, 0)),
          pl.BlockSpec((8, H), lambda i: (0, 0)),
          pl.BlockSpec((H, D), lambda i: (0, 0)),
          pl.BlockSpec((1, D), lambda i: (0, 0)),
      ],
      out_specs=pl.BlockSpec((blk, D), lambda i: (i, 0)),
      out_shape=jax.ShapeDtypeStruct((B, D), jnp.float32),
  )(xb, xs, num_aug, w1b, w1s, w1_aug, w2, b2_2d)


def kernel(cat0, cat1, cat2, cat3, cat4, cat5, numerical_inputs,
           T0, T1, T2, T3, T4, T5, W1, b1, W2, b2):
  cb = [c.astype(jnp.int32) for c in (cat0, cat1)]
  cs = [c.astype(jnp.int32).reshape(NW * NCH, CH)
        for c in (cat2, cat3, cat4, cat5)]
  tt0 = T0.T.reshape(4, 8, VBIG)
  tt1 = T1.T.reshape(4, 8, VBIG)
  packed, pos = _sc_scan_big(cb[0], cb[1], tt0, tt1)
  pk32 = packed.reshape(NBIG, NW, CAPP, D)
  emb_sml, emb_big = _sc_small_and_scatter(cs, [T2, T3, T4, T5], pk32, pos)

  ones = jnp.ones((B, 1), jnp.float32)
  zeros = jnp.zeros((B, 3), jnp.float32)
  num_aug = jnp.concatenate([numerical_inputs, ones, zeros], axis=1)
  w1_aug = jnp.concatenate(
      [W1[NT * D:], b1[None, :], jnp.zeros((3, H), jnp.float32)], axis=0)
  w1b = W1[:NBIG * D].reshape(NBIG, D, H)
  w1s = W1[NBIG * D:NT * D].reshape(NSML, D, H)
  return _tc_mlp(emb_big, emb_sml, num_aug, w1b, w1s, w1_aug, W2,
                 b2[None, :])


# speedup vs baseline: 1.5178x; 1.5178x over previous
import functools

import jax
import jax.numpy as jnp
from jax import lax
from jax.experimental import pallas as pl
from jax.experimental.pallas import tpu as pltpu
from jax.experimental.pallas import tpu_sc as plsc

B = 16384
D = 32
H = 256
NT = 6
NBIG = 2
NSML = 4
NC, NS = 2, 16
NW = NC * NS
BPW = B // NW
CH = 128
NCH = BPW // CH
NRING = 12


def _sc_gather_big(cat0, cat1, tt0, tt1):
  mesh = plsc.VectorSubcoreMesh(core_axis_name="c", subcore_axis_name="s")

  @functools.partial(
      pl.kernel,
      out_type=jax.ShapeDtypeStruct((NBIG, B, D), jnp.float32),
      mesh=mesh,
      compiler_params=pltpu.CompilerParams(
          use_tc_tiling_on_sc=True, needs_layout_passes=False),
      scratch_types=[
          pltpu.VMEM((BPW + 16,), jnp.int32),
          pltpu.VMEM((NRING, 4, 8, 128), jnp.float32),
          pltpu.VMEM((BPW, D), jnp.float32),
          pltpu.SemaphoreType.DMA,
      ],
  )
  def k(c0, c1, t0, t1, out, idx_v, tile_v, emb_v, sem):
    wid = lax.axis_index("s") * NC + lax.axis_index("c")
    base = wid * BPW
    d16 = lax.iota(jnp.int32, 16)
    ga, ra = d16 // 8, d16 % 8
    gb, rb = ga + 2, ra

    for t, (cat, tab) in enumerate(((c0, t0), (c1, t1))):
      pltpu.sync_copy(cat.at[pl.ds(base, BPW)], idx_v.at[pl.ds(0, BPW)])

      def fire(b, tab=tab):
        v = idx_v[pl.ds(b, 16)][0]
        pltpu.async_copy(
            tab.at[:, :, pl.ds((v // 128) * 128, 128)],
            tile_v.at[lax.rem(b, NRING)], sem)

      for b0 in range(NRING):
        fire(b0)

      def body(b, carry, tab=tab):
        slot = lax.rem(b, NRING)
        pltpu.make_async_copy(
            tab.at[:, :, pl.ds(0, 128)], tile_v.at[slot], sem).wait()
        v = idx_v[pl.ds(b, 16)][0]
        lane = jnp.full((16,), lax.rem(v, 128), jnp.int32)
        bsp = jnp.full((16,), b, jnp.int32)
        va = plsc.load_gather(tile_v.at[slot], [ga, ra, lane])
        vb = plsc.load_gather(tile_v.at[slot], [gb, rb, lane])
        plsc.store_scatter(emb_v, [bsp, d16], va)
        plsc.store_scatter(emb_v, [bsp, d16 + 16], vb)

        @pl.when(b + NRING < BPW)
        def _():
          fire(b + NRING)

        return carry

      lax.fori_loop(0, BPW, body, 0)
      pltpu.sync_copy(emb_v, out.at[t, pl.ds(base, BPW)])

  return k(cat0, cat1, tt0, tt1)


def _sc_gather_small(cats2d, tables):
  mesh = plsc.VectorSubcoreMesh(core_axis_name="c", subcore_axis_name="s")

  @functools.partial(
      pl.kernel,
      out_type=jax.ShapeDtypeStruct((NSML, B, D), jnp.float32),
      mesh=mesh,
      compiler_params=pltpu.CompilerParams(use_tc_tiling_on_sc=False),
      scratch_types=[
          pltpu.VMEM((NSML * NCH, CH), jnp.int32),
          pltpu.VMEM((NSML, BPW, D), jnp.float32),
          pltpu.SemaphoreType.DMA,
      ],
  )
  def k(c0, c1, c2, c3, t0, t1, t2, t3, out, idx_v, rows_v, sem):
    wid = lax.axis_index("s") * NC + lax.axis_index("c")
    base = wid * BPW
    cats = [c0, c1, c2, c3]
    tabs = [t0, t1, t2, t3]
    copies = []
    for i in range(NSML):
      pltpu.sync_copy(cats[i].at[pl.ds(wid * NCH, NCH)],
                      idx_v.at[pl.ds(i * NCH, NCH)])
      for j in range(NCH):
        copies.append(pltpu.async_copy(
            tabs[i].at[idx_v.at[i * NCH + j]],
            rows_v.at[i, pl.ds(j * CH, CH)], sem))
    for i in range(NSML):
      for j in range(NCH):
        copies[i * NCH + j].wait()
      pltpu.sync_copy(rows_v.at[i], out.at[i, pl.ds(base, BPW)])

  return k(*cats2d, *tables)


def _mlp_body(xb_ref, xs_ref, n_ref, w1b_ref, w1s_ref, wa_ref, w2_ref,
              b2_ref, o_ref):
  h = jnp.dot(n_ref[...], wa_ref[...], preferred_element_type=jnp.float32)
  for t in range(NBIG):
    h = h + jnp.dot(xb_ref[t], w1b_ref[t],
                    preferred_element_type=jnp.float32)
  for t in range(NSML):
    h = h + jnp.dot(xs_ref[t], w1s_ref[t],
                    preferred_element_type=jnp.float32)
  h = jnp.maximum(h, 0.0)
  o_ref[...] = (
      jnp.dot(h, w2_ref[...], preferred_element_type=jnp.float32)
      + b2_ref[...]
  )


def _tc_mlp(xb, xs, num_aug, w1b, w1s, w1_aug, w2, b2_2d):
  blk = 4096
  nb = B // blk
  return pl.pallas_call(
      _mlp_body,
      grid=(nb,),
      in_specs=[
          pl.BlockSpec((NBIG, blk, D), lambda i: (0, i, 0)),
          pl.BlockSpec((NSML, blk, D), lambda i: (0, i, 0)),
          pl.BlockSpec((blk, 8), lambda i: (i, 0)),
          pl.BlockSpec((NBIG, D, H), lambda i: (0, 0, 0)),
          pl.BlockSpec((NSML, D, H), lambda i: (0, 0, 0)),
          pl.BlockSpec((8, H), lambda i: (0, 0)),
          pl.BlockSpec((H, D), lambda i: (0, 0)),
          pl.BlockSpec((1, D), lambda i: (0, 0)),
      ],
      out_specs=pl.BlockSpec((blk, D), lambda i: (i, 0)),
      out_shape=jax.ShapeDtypeStruct((B, D), jnp.float32),
  )(xb, xs, num_aug, w1b, w1s, w1_aug, w2, b2_2d)


def kernel(cat0, cat1, cat2, cat3, cat4, cat5, numerical_inputs,
           T0, T1, T2, T3, T4, T5, W1, b1, W2, b2):
  cb = [c.astype(jnp.int32) for c in (cat0, cat1)]
  cs = [c.astype(jnp.int32).reshape(NW * NCH, CH)
        for c in (cat2, cat3, cat4, cat5)]
  tt0 = T0.T.reshape(4, 8, T0.shape[0])
  tt1 = T1.T.reshape(4, 8, T1.shape[0])
  emb_big = _sc_gather_big(cb[0], cb[1], tt0, tt1)
  emb_sml = _sc_gather_small(cs, [T2, T3, T4, T5])

  ones = jnp.ones((B, 1), jnp.float32)
  zeros = jnp.zeros((B, 3), jnp.float32)
  num_aug = jnp.concatenate([numerical_inputs, ones, zeros], axis=1)
  w1_aug = jnp.concatenate(
      [W1[NT * D:], b1[None, :], jnp.zeros((3, H), jnp.float32)], axis=0)
  w1b = W1[:NBIG * D].reshape(NBIG, D, H)
  w1s = W1[NBIG * D:NT * D].reshape(NSML, D, H)
  return _tc_mlp(emb_big, emb_sml, num_aug, w1b, w1s, w1_aug, W2,
                 b2[None, :])
